# Initial kernel scaffold; baseline (speedup 1.0000x reference)
import jax
import jax.numpy as jnp
from jax.experimental import pallas as pl


def kernel(padrao, erro, feedback_gain):
    raise NotImplementedError("write your pallas kernel here")


# speedup vs baseline: 16.5222x; 16.5222x over previous
import jax
import jax.numpy as jnp
from jax import lax
from jax.experimental import pallas as pl
from jax.experimental.pallas import tpu as pltpu

_B = 512
_D = 64
_LR = 0.01
_THR = 0.7
_EPS = 1e-8


def _engram_body(padrao_ref, gain_ref, out_ref, protos_ref):
    gain = gain_ref[0, 0]
    iota = lax.broadcasted_iota(jnp.int32, (1, _B), 1)
    neg_inf = jnp.float32(-jnp.inf)

    def step(t, carry):
        n, norm_vec, forca_vec = carry
        p_row = padrao_ref[pl.ds(t, 1), :]
        p_norm = jnp.maximum(jnp.sqrt(jnp.sum(p_row * p_row)), _EPS)
        dots = lax.dot_general(
            p_row,
            protos_ref[:, :],
            (((1,), (1,)), ((), ())),
            preferred_element_type=jnp.float32,
            precision=lax.Precision.HIGHEST,
        )
        sims = dots / (norm_vec * p_norm)
        masked = jnp.where(iota < n, sims, neg_inf)
        max_sim = jnp.max(masked)
        idx = jnp.min(jnp.where(masked == max_sim, iota, _B))
        reinforce = jnp.logical_and(n > 0, max_sim >= _THR)
        w = jnp.where(reinforce, idx, n)

        old_row = protos_ref[pl.ds(w, 1), :]
        new_row = jnp.where(reinforce, (1.0 - _LR) * old_row + _LR * p_row, p_row)
        f_idx = jnp.sum(jnp.where(iota == idx, forca_vec, 0.0))
        new_forca = jnp.where(reinforce, f_idx + _LR, 1.0)
        fb_row = jnp.where(reinforce, (new_row - p_row) * new_forca, 0.0) * gain

        out_ref[pl.ds(t, 1), :] = fb_row
        protos_ref[pl.ds(w, 1), :] = new_row
        new_norm = jnp.maximum(jnp.sqrt(jnp.sum(new_row * new_row)), _EPS)
        norm_vec = jnp.where(iota == w, new_norm, norm_vec)
        forca_vec = jnp.where(iota == w, new_forca, forca_vec)
        n = n + jnp.where(reinforce, 0, 1)
        return n, norm_vec, forca_vec

    lax.fori_loop(
        0,
        _B,
        step,
        (
            jnp.int32(0),
            jnp.ones((1, _B), jnp.float32),
            jnp.zeros((1, _B), jnp.float32),
        ),
    )


def kernel(padrao, erro, feedback_gain):
    del erro
    gain2d = jnp.reshape(feedback_gain.astype(jnp.float32), (1, 1))
    return pl.pallas_call(
        _engram_body,
        out_shape=jax.ShapeDtypeStruct((_B, _D), jnp.float32),
        in_specs=[
            pl.BlockSpec(memory_space=pltpu.MemorySpace.VMEM),
            pl.BlockSpec(memory_space=pltpu.MemorySpace.SMEM),
        ],
        out_specs=pl.BlockSpec(memory_space=pltpu.MemorySpace.VMEM),
        scratch_shapes=[pltpu.VMEM((_B, _D), jnp.float32)],
    )(padrao, gain2d)


# speedup vs baseline: 21.4814x; 1.3001x over previous
import functools

import jax
import jax.numpy as jnp
from jax import lax
from jax.experimental import pallas as pl
from jax.experimental.pallas import tpu as pltpu
from jax.experimental.pallas import tpu_sc as plsc

_B = 512
_D = 64
_NT = 16
_RPT = 32
_LR = 0.01
_EPS2 = 1e-16


def _sc_body(padrao_hbm, gain_hbm, out_hbm, pad_v, prot_v, stage_v, cands_v,
             fb_v, zero_v, gain_v, shared):
    cid = lax.axis_index("c")
    sid = lax.axis_index("s")

    @pl.when(cid == 0)
    def _core0():
        k = sid
        iota = lax.iota(jnp.int32, 16)
        zero16 = jnp.zeros((16,), jnp.float32)
        one16 = jnp.full((16,), 1, jnp.int32)
        neg_inf = jnp.float32(-jnp.inf)

        pltpu.sync_copy(padrao_hbm, pad_v)
        pltpu.sync_copy(gain_hbm, gain_v)
        for r in range(_RPT):
            for c in range(4):
                zero_v[r, pl.ds(16 * c, 16)] = zero16
        pltpu.sync_copy(zero_v, out_hbm.at[pl.ds(k * _RPT, _RPT)])
        gain = gain_v[:]

        def halfstep(t, buf, carry):
            n, sq0, sq1, f0, f1 = carry
            t_vec = jnp.full((16,), t, jnp.int32)
            pc = [pad_v[t, pl.ds(16 * c, 16)] for c in range(4)]
            psq_vec = pc[0] * pc[0] + pc[1] * pc[1] + pc[2] * pc[2] + pc[3] * pc[3]
            p_sq = jnp.sum(psq_vec)

            acc0 = zero16
            acc1 = zero16
            for d in range(_D):
                pd = plsc.load_gather(pad_v, [t_vec, jnp.full((16,), d, jnp.int32)])
                acc0 = acc0 + prot_v[d, 0, :] * pd
                acc1 = acc1 + prot_v[d, 1, :] * pd

            b0 = acc0 * jnp.abs(acc0) / jnp.maximum(sq0, _EPS2)
            b1 = acc1 * jnp.abs(acc1) / jnp.maximum(sq1, _EPS2)
            valid0 = (iota * 16 + k) < n
            valid1 = (iota * 16 + k + 256) < n
            m0 = jnp.where(valid0, b0, neg_inf)
            m1 = jnp.where(valid1, b1, neg_inf)
            mloc = jnp.maximum(jnp.max(m0), jnp.max(m1))
            j0 = jnp.min(jnp.where(m0 == mloc, iota, 999))
            j1 = jnp.min(jnp.where(m1 == mloc, iota + 16, 999))
            gidx = jnp.minimum(j0, j1) * 16 + k

            cand = jnp.where(iota == 0, jnp.full((16,), mloc),
                             jnp.where(iota == 1,
                                       plsc.bitcast(jnp.full((16,), gidx), jnp.float32),
                                       zero16))
            stage_v[:] = cand
            pltpu.sync_copy(stage_v, shared.at[buf, k])
            plsc.subcore_barrier()
            pltpu.sync_copy(shared.at[buf], cands_v)
            vals = plsc.load_gather(cands_v, [iota, jnp.zeros((16,), jnp.int32)])
            idxs = plsc.bitcast(plsc.load_gather(cands_v, [iota, one16]), jnp.int32)
            mg = jnp.max(vals)
            wc = jnp.min(jnp.where(vals == mg, idxs, 9999))
            reinforce = jnp.logical_and(n > 0, mg >= 0.49 * jnp.maximum(p_sq, _EPS2))
            w = jnp.where(reinforce, wc, n)

            mine = lax.rem(w, 16) == k
            j = lax.div(w, 16)
            g = lax.div(j, 16)
            l = lax.rem(j, 16)
            g_vec = jnp.full((16,), g, jnp.int32)
            l_vec = jnp.full((16,), l, jnp.int32)
            mask = jnp.full((16,), mine)
            old = [plsc.load_gather(prot_v, [iota + 16 * c, g_vec, l_vec])
                   for c in range(4)]
            new = [jnp.where(reinforce, (1.0 - _LR) * old[c] + _LR * pc[c], pc[c])
                   for c in range(4)]
            for c in range(4):
                plsc.store_scatter(prot_v, [iota + 16 * c, g_vec, l_vec], new[c],
                                   mask=mask)
            nsq_vec = (new[0] * new[0] + new[1] * new[1] + new[2] * new[2]
                       + new[3] * new[3])
            new_sq = jnp.sum(nsq_vec)
            fsel0 = jnp.sum(jnp.where(iota == l, f0, 0.0))
            fsel1 = jnp.sum(jnp.where(iota == l, f1, 0.0))
            f_old = jnp.where(g == 0, fsel0, fsel1)
            new_f = jnp.where(reinforce, f_old + _LR, 1.0)

            sel0 = jnp.logical_and(iota == l, jnp.full((16,), jnp.logical_and(mine, g == 0)))
            sel1 = jnp.logical_and(iota == l, jnp.full((16,), jnp.logical_and(mine, g == 1)))
            sq0n = jnp.where(sel0, jnp.full((16,), new_sq), sq0)
            sq1n = jnp.where(sel1, jnp.full((16,), new_sq), sq1)
            f0n = jnp.where(sel0, jnp.full((16,), new_f), f0)
            f1n = jnp.where(sel1, jnp.full((16,), new_f), f1)

            @pl.when(jnp.logical_and(reinforce, mine))
            def _emit():
                for c in range(4):
                    fb_v[pl.ds(16 * c, 16)] = (new[c] - pc[c]) * new_f * gain
                pltpu.sync_copy(fb_v, out_hbm.at[t])

            n_new = n + jnp.where(reinforce, 0, 1)
            return (n_new, sq0n, sq1n, f0n, f1n)

        def two_steps(i, carry):
            carry = halfstep(2 * i, 0, carry)
            carry = halfstep(2 * i + 1, 1, carry)
            return carry

        lax.fori_loop(0, _B // 2, two_steps,
                      (jnp.int32(0), zero16, zero16, zero16, zero16))


def _engram_sc(padrao, gain16):
    mesh = plsc.VectorSubcoreMesh(core_axis_name="c", subcore_axis_name="s")
    f = functools.partial(
        pl.kernel,
        out_type=jax.ShapeDtypeStruct((_B, _D), jnp.float32),
        mesh=mesh,
        scratch_types=[
            pltpu.VMEM((_B, _D), jnp.float32),
            pltpu.VMEM((_D, 2, 16), jnp.float32),
            pltpu.VMEM((16,), jnp.float32),
            pltpu.VMEM((16, 16), jnp.float32),
            pltpu.VMEM((_D,), jnp.float32),
            pltpu.VMEM((_RPT, _D), jnp.float32),
            pltpu.VMEM((16,), jnp.float32),
            pltpu.VMEM_SHARED((2, 16, 16), jnp.float32),
        ],
        compiler_params=pltpu.CompilerParams(needs_layout_passes=False),
    )(_sc_body)
    return f(padrao, gain16)


def kernel(padrao, erro, feedback_gain):
    del erro
    gain16 = jnp.broadcast_to(
        jnp.reshape(feedback_gain.astype(jnp.float32), (1,)), (16,))
    return _engram_sc(padrao, gain16)


# speedup vs baseline: 23.3103x; 1.0851x over previous
import functools

import jax
import jax.numpy as jnp
from jax import lax
from jax.experimental import pallas as pl
from jax.experimental.pallas import tpu as pltpu
from jax.experimental.pallas import tpu_sc as plsc

_B = 512
_D = 64
_RPT = 32
_LR = 0.01
_EPS2 = 1e-16

_GDN = lax.GatherDimensionNumbers(
    offset_dims=(), collapsed_slice_dims=(0,), start_index_map=(0,))


def _vtake(x, idx16):
    return lax.gather(x, idx16[:, None], _GDN, (1,),
                      mode=lax.GatherScatterMode.PROMISE_IN_BOUNDS)


def _sc_body(padrao_hbm, gain_hbm, out_hbm, pad_v, prot_v, stage_v, board_v,
             fb_v, zero_v, gain_v, sq_v, f_v, shared):
    cid = lax.axis_index("c")
    sid = lax.axis_index("s")

    @pl.when(cid == 0)
    def _core0():
        k = sid
        iota = lax.iota(jnp.int32, 16)
        zero16 = jnp.zeros((16,), jnp.float32)
        zero16i = jnp.zeros((16,), jnp.int32)
        one16i = jnp.full((16,), 1, jnp.int32)
        neg_inf = jnp.float32(-jnp.inf)

        pltpu.sync_copy(padrao_hbm, pad_v)
        pltpu.sync_copy(gain_hbm, gain_v)
        for r in range(_RPT):
            for c in range(4):
                zero_v[r, pl.ds(16 * c, 16)] = zero16
        pltpu.sync_copy(zero_v, out_hbm.at[pl.ds(k * _RPT, _RPT)])
        gain = gain_v[:]

        def halfstep(t, buf, carry, both_groups):
            n, inv0, inv1 = carry
            pc = [pad_v[t, pl.ds(16 * c, 16)] for c in range(4)]
            psq_vec = pc[0] * pc[0] + pc[1] * pc[1] + pc[2] * pc[2] + pc[3] * pc[3]
            p_sq = jnp.sum(psq_vec)

            t_vec = jnp.full((16,), t, jnp.int32)
            acc0 = zero16
            acc1 = zero16
            for d in range(_D):
                pd = plsc.load_gather(pad_v,
                                      [t_vec, jnp.full((16,), d, jnp.int32)])
                acc0 = acc0 + prot_v[d, 0, :] * pd
                if both_groups:
                    acc1 = acc1 + prot_v[d, 1, :] * pd

            b0 = acc0 * jnp.abs(acc0) * inv0
            valid0 = (iota * 16 + k) < n
            m0 = jnp.where(valid0, b0, neg_inf)
            if both_groups:
                b1 = acc1 * jnp.abs(acc1) * inv1
                valid1 = (iota * 16 + k + 256) < n
                m1 = jnp.where(valid1, b1, neg_inf)
                mloc = jnp.max(jnp.maximum(m0, m1))
                cvec = jnp.where(m0 == mloc, iota,
                                 jnp.where(m1 == mloc, iota + 16, 999))
            else:
                mloc = jnp.max(m0)
                cvec = jnp.where(m0 == mloc, iota, 999)
            jloc = jnp.min(cvec)
            gidx = jloc * 16 + k

            cand = jnp.where(iota == 0, jnp.full((16,), mloc),
                             plsc.bitcast(jnp.full((16,), gidx), jnp.float32))
            stage_v[:] = cand
            pltpu.sync_copy(stage_v, shared.at[buf, k])
            plsc.subcore_barrier()
            pltpu.sync_copy(shared.at[buf], board_v)
            vals = plsc.load_gather(board_v, [iota, zero16i])
            idxs = plsc.bitcast(plsc.load_gather(board_v, [iota, one16i]),
                                jnp.int32)
            mg = jnp.max(vals)
            wc = jnp.min(jnp.where(vals == mg, idxs, 9999))
            reinforce = jnp.logical_and(
                n > 0, mg >= 0.49 * jnp.maximum(p_sq, _EPS2))
            w = jnp.where(reinforce, wc, n)

            mine = lax.rem(w, 16) == k
            j = lax.div(w, 16)
            g = lax.div(j, 16)
            l = lax.rem(j, 16)
            g_vec = jnp.full((16,), g, jnp.int32)
            l_vec = jnp.full((16,), l, jnp.int32)
            j_vec = jnp.full((16,), j, jnp.int32)
            mine_vec = jnp.full((16,), mine)
            lane0_mine = jnp.logical_and(mine_vec, iota == 0)
            old = [plsc.load_gather(prot_v, [iota + 16 * c, g_vec, l_vec])
                   for c in range(4)]
            new = [jnp.where(reinforce, (1.0 - _LR) * old[c] + _LR * pc[c],
                             pc[c]) for c in range(4)]
            for c in range(4):
                plsc.store_scatter(prot_v, [iota + 16 * c, g_vec, l_vec],
                                   new[c], mask=mine_vec)

            if both_groups:
                acc_sel = jnp.where(jnp.full((16,), g == 0), acc0, acc1)
            else:
                acc_sel = acc0
            dot_vec = jnp.full((16,), jnp.sum(jnp.where(iota == l, acc_sel, 0.0)))
            sq_old = plsc.load_gather(sq_v, [j_vec])
            psq_b = jnp.full((16,), p_sq)
            sq_new = jnp.where(jnp.full((16,), reinforce),
                               (0.99 * 0.99) * sq_old + (2.0 * 0.99 * 0.01)
                               * dot_vec + (0.01 * 0.01) * psq_b,
                               psq_b)
            plsc.store_scatter(sq_v, [j_vec], sq_new, mask=lane0_mine)
            inv_new = 1.0 / jnp.maximum(sq_new, _EPS2)
            sel0 = jnp.logical_and(
                jnp.logical_and(mine_vec, jnp.full((16,), g == 0)), iota == l)
            sel1 = jnp.logical_and(
                jnp.logical_and(mine_vec, jnp.full((16,), g == 1)), iota == l)
            inv0n = jnp.where(sel0, inv_new, inv0)
            inv1n = jnp.where(sel1, inv_new, inv1)

            f_old = plsc.load_gather(f_v, [j_vec])
            new_f = jnp.where(jnp.full((16,), reinforce), f_old + _LR, 1.0)
            plsc.store_scatter(f_v, [j_vec], new_f, mask=lane0_mine)

            @pl.when(jnp.logical_and(reinforce, mine))
            def _emit():
                for c in range(4):
                    fb_v[pl.ds(16 * c, 16)] = (new[c] - pc[c]) * new_f * gain
                pltpu.sync_copy(fb_v, out_hbm.at[t])

            n_new = n + jnp.where(reinforce, 0, 1)
            return (n_new, inv0n, inv1n)

        def two_steps_g0(i, carry):
            carry = halfstep(2 * i, 0, carry, False)
            carry = halfstep(2 * i + 1, 1, carry, False)
            return carry

        def two_steps_g01(i, carry):
            carry = halfstep(2 * i, 0, carry, True)
            carry = halfstep(2 * i + 1, 1, carry, True)
            return carry

        ones16 = jnp.ones((16,), jnp.float32)
        carry = lax.fori_loop(0, _B // 4, two_steps_g0,
                              (jnp.int32(0), ones16, ones16))
        lax.fori_loop(_B // 4, _B // 2, two_steps_g01, carry)


def _engram_sc(padrao, gain16):
    mesh = plsc.VectorSubcoreMesh(core_axis_name="c", subcore_axis_name="s")
    f = functools.partial(
        pl.kernel,
        out_type=jax.ShapeDtypeStruct((_B, _D), jnp.float32),
        mesh=mesh,
        scratch_types=[
            pltpu.VMEM((_B, _D), jnp.float32),
            pltpu.VMEM((_D, 2, 16), jnp.float32),
            pltpu.VMEM((16,), jnp.float32),
            pltpu.VMEM((16, 16), jnp.float32),
            pltpu.VMEM((_D,), jnp.float32),
            pltpu.VMEM((_RPT, _D), jnp.float32),
            pltpu.VMEM((16,), jnp.float32),
            pltpu.VMEM((_RPT,), jnp.float32),
            pltpu.VMEM((_RPT,), jnp.float32),
            pltpu.VMEM_SHARED((2, 16, 16), jnp.float32),
        ],
        compiler_params=pltpu.CompilerParams(needs_layout_passes=False),
    )(_sc_body)
    return f(padrao, gain16)


def kernel(padrao, erro, feedback_gain):
    del erro
    gain16 = jnp.broadcast_to(
        jnp.reshape(feedback_gain.astype(jnp.float32), (1,)), (16,))
    return _engram_sc(padrao, gain16)


# speedup vs baseline: 23.5966x; 1.0123x over previous
import functools

import jax
import jax.numpy as jnp
from jax import lax
from jax.experimental import pallas as pl
from jax.experimental.pallas import tpu as pltpu
from jax.experimental.pallas import tpu_sc as plsc

_B = 512
_D = 64
_RPT = 32
_LR = 0.01
_EPS2 = 1e-16

_GDN = lax.GatherDimensionNumbers(
    offset_dims=(), collapsed_slice_dims=(0,), start_index_map=(0,))


def _vtake(x, idx16):
    return lax.gather(x, idx16[:, None], _GDN, (1,),
                      mode=lax.GatherScatterMode.PROMISE_IN_BOUNDS)


def _sc_body(padrao_hbm, gain_hbm, out_hbm, pad_v, prot_v, stage_v, board_v,
             fb_v, zero_v, gain_v, sq_v, f_v, shared):
    cid = lax.axis_index("c")
    sid = lax.axis_index("s")

    @pl.when(cid == 0)
    def _core0():
        k = sid
        iota = lax.iota(jnp.int32, 16)
        zero16 = jnp.zeros((16,), jnp.float32)
        zero16i = jnp.zeros((16,), jnp.int32)
        one16i = jnp.full((16,), 1, jnp.int32)
        neg_inf = jnp.float32(-jnp.inf)

        pltpu.sync_copy(padrao_hbm, pad_v)
        pltpu.sync_copy(gain_hbm, gain_v)
        for r in range(_RPT):
            for c in range(4):
                zero_v[r, pl.ds(16 * c, 16)] = zero16
        pltpu.sync_copy(zero_v, out_hbm.at[pl.ds(k * _RPT, _RPT)])
        gain = gain_v[:]

        def halfstep(t, buf, carry, both_groups):
            n, inv0, inv1 = carry
            pc = [pad_v[t, pl.ds(16 * c, 16)] for c in range(4)]
            psq_vec = pc[0] * pc[0] + pc[1] * pc[1] + pc[2] * pc[2] + pc[3] * pc[3]
            p_sq = jnp.sum(psq_vec)

            t_vec = jnp.full((16,), t, jnp.int32)
            acc0 = zero16
            acc1 = zero16
            for d in range(_D):
                pd = plsc.load_gather(pad_v,
                                      [t_vec, jnp.full((16,), d, jnp.int32)])
                acc0 = acc0 + prot_v[d, 0, :] * pd
                if both_groups:
                    acc1 = acc1 + prot_v[d, 1, :] * pd

            b0 = acc0 * jnp.abs(acc0) * inv0
            valid0 = (iota * 16 + k) < n
            m0 = jnp.where(valid0, b0, neg_inf)
            if both_groups:
                b1 = acc1 * jnp.abs(acc1) * inv1
                valid1 = (iota * 16 + k + 256) < n
                m1 = jnp.where(valid1, b1, neg_inf)
                mloc = jnp.max(jnp.maximum(m0, m1))
                cvec = jnp.where(m0 == mloc, iota,
                                 jnp.where(m1 == mloc, iota + 16, 999))
            else:
                mloc = jnp.max(m0)
                cvec = jnp.where(m0 == mloc, iota, 999)
            jloc = jnp.min(cvec)
            gidx = jloc * 16 + k

            cand = jnp.where(iota == 0, jnp.full((16,), mloc),
                             plsc.bitcast(jnp.full((16,), gidx), jnp.float32))
            stage_v[:] = cand
            pltpu.sync_copy(stage_v.at[pl.ds(0, 2)], shared.at[buf, k])
            plsc.subcore_barrier()
            pltpu.sync_copy(shared.at[buf], board_v)
            vals = plsc.load_gather(board_v, [iota, zero16i])
            idxs = plsc.bitcast(plsc.load_gather(board_v, [iota, one16i]),
                                jnp.int32)
            mg = jnp.max(vals)
            wc = jnp.min(jnp.where(vals == mg, idxs, 9999))
            reinforce = jnp.logical_and(
                n > 0, mg >= 0.49 * jnp.maximum(p_sq, _EPS2))
            w = jnp.where(reinforce, wc, n)

            mine = lax.rem(w, 16) == k
            j = lax.div(w, 16)
            g = lax.div(j, 16)
            l = lax.rem(j, 16)
            g_vec = jnp.full((16,), g, jnp.int32)
            l_vec = jnp.full((16,), l, jnp.int32)
            j_vec = jnp.full((16,), j, jnp.int32)
            mine_vec = jnp.full((16,), mine)
            lane0_mine = jnp.logical_and(mine_vec, iota == 0)
            old = [plsc.load_gather(prot_v, [iota + 16 * c, g_vec, l_vec])
                   for c in range(4)]
            new = [jnp.where(reinforce, (1.0 - _LR) * old[c] + _LR * pc[c],
                             pc[c]) for c in range(4)]
            for c in range(4):
                plsc.store_scatter(prot_v, [iota + 16 * c, g_vec, l_vec],
                                   new[c], mask=mine_vec)

            if both_groups:
                acc_sel = jnp.where(jnp.full((16,), g == 0), acc0, acc1)
            else:
                acc_sel = acc0
            dot_vec = jnp.full((16,), jnp.sum(jnp.where(iota == l, acc_sel, 0.0)))
            sq_old = plsc.load_gather(sq_v, [j_vec])
            psq_b = jnp.full((16,), p_sq)
            sq_new = jnp.where(jnp.full((16,), reinforce),
                               (0.99 * 0.99) * sq_old + (2.0 * 0.99 * 0.01)
                               * dot_vec + (0.01 * 0.01) * psq_b,
                               psq_b)
            plsc.store_scatter(sq_v, [j_vec], sq_new, mask=lane0_mine)
            inv_new = 1.0 / jnp.maximum(sq_new, _EPS2)
            sel0 = jnp.logical_and(
                jnp.logical_and(mine_vec, jnp.full((16,), g == 0)), iota == l)
            sel1 = jnp.logical_and(
                jnp.logical_and(mine_vec, jnp.full((16,), g == 1)), iota == l)
            inv0n = jnp.where(sel0, inv_new, inv0)
            inv1n = jnp.where(sel1, inv_new, inv1)

            f_old = plsc.load_gather(f_v, [j_vec])
            new_f = jnp.where(jnp.full((16,), reinforce), f_old + _LR, 1.0)
            plsc.store_scatter(f_v, [j_vec], new_f, mask=lane0_mine)

            @pl.when(jnp.logical_and(reinforce, mine))
            def _emit():
                for c in range(4):
                    fb_v[pl.ds(16 * c, 16)] = (new[c] - pc[c]) * new_f * gain
                pltpu.sync_copy(fb_v, out_hbm.at[t])

            n_new = n + jnp.where(reinforce, 0, 1)
            return (n_new, inv0n, inv1n)

        def two_steps_g0(i, carry):
            carry = halfstep(2 * i, 0, carry, False)
            carry = halfstep(2 * i + 1, 1, carry, False)
            return carry

        def two_steps_g01(i, carry):
            carry = halfstep(2 * i, 0, carry, True)
            carry = halfstep(2 * i + 1, 1, carry, True)
            return carry

        ones16 = jnp.ones((16,), jnp.float32)
        carry = lax.fori_loop(0, _B // 4, two_steps_g0,
                              (jnp.int32(0), ones16, ones16))
        lax.fori_loop(_B // 4, _B // 2, two_steps_g01, carry)


def _engram_sc(padrao, gain16):
    mesh = plsc.VectorSubcoreMesh(core_axis_name="c", subcore_axis_name="s")
    f = functools.partial(
        pl.kernel,
        out_type=jax.ShapeDtypeStruct((_B, _D), jnp.float32),
        mesh=mesh,
        scratch_types=[
            pltpu.VMEM((_B, _D), jnp.float32),
            pltpu.VMEM((_D, 2, 16), jnp.float32),
            pltpu.VMEM((16,), jnp.float32),
            pltpu.VMEM((16, 2), jnp.float32),
            pltpu.VMEM((_D,), jnp.float32),
            pltpu.VMEM((_RPT, _D), jnp.float32),
            pltpu.VMEM((16,), jnp.float32),
            pltpu.VMEM((_RPT,), jnp.float32),
            pltpu.VMEM((_RPT,), jnp.float32),
            pltpu.VMEM_SHARED((2, 16, 2), jnp.float32),
        ],
        compiler_params=pltpu.CompilerParams(needs_layout_passes=False),
    )(_sc_body)
    return f(padrao, gain16)


def kernel(padrao, erro, feedback_gain):
    del erro
    gain16 = jnp.broadcast_to(
        jnp.reshape(feedback_gain.astype(jnp.float32), (1,)), (16,))
    return _engram_sc(padrao, gain16)
